# baseline (speedup 1.0000x reference)
import functools

import jax
import jax.numpy as jnp
from jax import lax
from jax.experimental import pallas as pl
from jax.experimental.pallas import tpu as pltpu
from jax.experimental.pallas import tpu_sc as plsc

BATCH = 16384
NF = 32
NC, NS = 2, 16
NW = NC * NS
B_PER_W = BATCH // NW
IC = 128
N_IC = B_PER_W // IC


NU = 1000000
NUP = 1000064
DCH = 65536
NDC = 15
TAIL = NUP - NDC * DCH


def _tc_detile(Pt, Qt, Ut, Vt):
    out = jax.ShapeDtypeStruct((NF * NUP,), jnp.float32)

    def body(pt, qt, ut, vt, op, oq, ou, ov, sem):
        pairs = [(pt, op), (qt, oq), (ut, ou), (vt, ov)]

        def fire(f, u0, n):
            for tbl, dst in pairs:
                pltpu.make_async_copy(
                    tbl.at[f, pl.ds(u0, n)],
                    dst.at[pl.ds(f * NUP + u0, n)], sem).start()

        def drain(n):
            for tbl, dst in pairs:
                pltpu.make_async_copy(
                    tbl.at[0, pl.ds(0, n)],
                    dst.at[pl.ds(0, n)], sem).wait()

        def step(i, carry):
            fire(i // NDC, (i % NDC) * DCH, DCH)

            @pl.when(i > 0)
            def _():
                drain(DCH)

            return carry

        lax.fori_loop(0, NF * NDC, step, 0)
        drain(DCH)

        def tail_step(f, carry):
            fire(f, pl.multiple_of(f * 0 + NDC * DCH, 128), TAIL)

            @pl.when(f > 0)
            def _():
                drain(TAIL)

            return carry

        lax.fori_loop(0, NF, tail_step, 0)
        drain(TAIL)

    any_spec = pl.BlockSpec(memory_space=pl.ANY)
    return pl.pallas_call(
        body,
        in_specs=[any_spec] * 4,
        out_specs=[any_spec] * 4,
        out_shape=(out, out, out, out),
        scratch_shapes=[pltpu.SemaphoreType.DMA],
        compiler_params=pltpu.CompilerParams(disable_bounds_checks=True),
    )(Pt, Qt, Ut, Vt)


def _sc_gather_t(Pl, Ql, Ul, Vl, user_id, item_id):
    mesh = plsc.VectorSubcoreMesh(core_axis_name="c", subcore_axis_name="s")
    out = jax.ShapeDtypeStruct((NF, BATCH), jnp.float32)

    @functools.partial(
        pl.kernel,
        mesh=mesh,
        out_type=(out, out, out, out),
        compiler_params=pltpu.CompilerParams(
            use_tc_tiling_on_sc=False, needs_layout_passes=False),
        scratch_types=[
            pltpu.VMEM((B_PER_W,), jnp.int32),
            pltpu.VMEM((B_PER_W,), jnp.int32),
            pltpu.VMEM((B_PER_W,), jnp.int32),
            pltpu.VMEM((B_PER_W,), jnp.int32),
            pltpu.VMEM((NF, B_PER_W), jnp.float32),
            pltpu.VMEM((NF, B_PER_W), jnp.float32),
            pltpu.VMEM((NF, B_PER_W), jnp.float32),
            pltpu.VMEM((NF, B_PER_W), jnp.float32),
            pltpu.SemaphoreType.DMA,
            pltpu.SemaphoreType.DMA,
            pltpu.SemaphoreType.DMA,
            pltpu.SemaphoreType.DMA,
        ],
    )
    def k(p_hbm, q_hbm, u_hbm, v_hbm, iu_hbm, ii_hbm,
          pmf_hbm, qmf_hbm, pml_hbm, qml_hbm,
          iu_v, ii_v, eu_v, ei_v, pv, qv, uv, vv, sp, sq, su, sv):
        wid = lax.axis_index("s") * NC + lax.axis_index("c")
        base = wid * B_PER_W
        pltpu.sync_copy(iu_hbm.at[pl.ds(base, B_PER_W)], iu_v)
        pltpu.sync_copy(ii_hbm.at[pl.ds(base, B_PER_W)], ii_v)

        @pl.loop(0, NF)
        def _(f):
            off = f * NUP

            @pl.loop(0, B_PER_W // 16)
            def _(cc):
                s = pl.ds(cc * 16, 16)
                eu_v[s] = iu_v[s] + off
                ei_v[s] = ii_v[s] + off

            pltpu.async_copy(p_hbm.at[eu_v], pv.at[f], sp)
            pltpu.async_copy(q_hbm.at[ei_v], qv.at[f], sq)
            pltpu.async_copy(u_hbm.at[eu_v], uv.at[f], su)
            pltpu.async_copy(v_hbm.at[ei_v], vv.at[f], sv)
            pltpu.make_async_copy(p_hbm.at[eu_v], pv.at[f], sp).wait()
            pltpu.make_async_copy(q_hbm.at[ei_v], qv.at[f], sq).wait()
            pltpu.make_async_copy(u_hbm.at[eu_v], uv.at[f], su).wait()
            pltpu.make_async_copy(v_hbm.at[ei_v], vv.at[f], sv).wait()

        pltpu.sync_copy(pv, pmf_hbm.at[:, pl.ds(base, B_PER_W)])
        pltpu.sync_copy(qv, qmf_hbm.at[:, pl.ds(base, B_PER_W)])
        pltpu.sync_copy(uv, pml_hbm.at[:, pl.ds(base, B_PER_W)])
        pltpu.sync_copy(vv, qml_hbm.at[:, pl.ds(base, B_PER_W)])

    return k(Pl, Ql, Ul, Vl, user_id, item_id)


def _dotT(w, x):
    return lax.dot_general(w, x, (((0,), (0,)), ((), ())),
                           preferred_element_type=jnp.float32)


def _tc_mlp_body(pmf_ref, qmf_ref, pml_ref, qml_ref,
                 w0_ref, b0_ref, w1_ref, b1_ref, w2_ref, b2_ref,
                 wp_ref, bp_ref, out_ref):
    h = (_dotT(w0_ref[:NF, :], pml_ref[...])
         + _dotT(w0_ref[NF:, :], qml_ref[...])
         + b0_ref[...].T)
    h = jnp.maximum(h, 0.0)
    h = _dotT(w1_ref[...], h) + b1_ref[...].T
    h = jnp.maximum(h, 0.0)
    h = _dotT(w2_ref[...], h) + b2_ref[...].T
    h = jnp.maximum(h, 0.0)
    gmf = pmf_ref[...] * qmf_ref[...]
    out = (_dotT(wp_ref[:NF, :], gmf)
           + _dotT(wp_ref[NF:, :], h)
           + bp_ref[...].T)
    out_ref[...] = out


def _tc_mlp(pmf, qmf, pml, qml, W0, b0, W1, b1, W2, b2, Wp, bp):
    blk = 4096
    grid = (BATCH // blk,)
    in_col = pl.BlockSpec((NF, blk), lambda i: (0, i))
    full = lambda a: pl.BlockSpec(a.shape, lambda i: (0,) * a.ndim)
    return pl.pallas_call(
        _tc_mlp_body,
        grid=grid,
        in_specs=[in_col, in_col, in_col, in_col,
                  full(W0), full(b0), full(W1), full(b1),
                  full(W2), full(b2), full(Wp), full(bp)],
        out_specs=pl.BlockSpec((1, blk), lambda i: (0, i)),
        out_shape=jax.ShapeDtypeStruct((1, BATCH), jnp.float32),
    )(pmf, qmf, pml, qml, W0, b0, W1, b1, W2, b2, Wp, bp)


def kernel(user_id, item_id, P, Q, U, V, W0, b0, W1, b1, W2, b2, Wp, bp):
    Pl, Ql, Ul, Vl = _tc_detile(P.T, Q.T, U.T, V.T)
    pmf, qmf, pml, qml = _sc_gather_t(Pl, Ql, Ul, Vl, user_id, item_id)
    out = _tc_mlp(pmf, qmf, pml, qml,
                  W0, b0.reshape(1, -1), W1, b1.reshape(1, -1),
                  W2, b2.reshape(1, -1), Wp, bp.reshape(1, -1))
    return out.reshape(BATCH)


# speedup vs baseline: 31.9999x; 31.9999x over previous
import functools

import jax
import jax.numpy as jnp
from jax import lax
from jax.experimental import pallas as pl
from jax.experimental.pallas import tpu as pltpu
from jax.experimental.pallas import tpu_sc as plsc

BATCH = 16384
NF = 32
NC, NS = 2, 16
NW = NC * NS
B_PER_W = BATCH // NW
IC = 128
N_IC = B_PER_W // IC


NU = 1000000
NUP = 1000064
DT_CH = 4096
DT_FULL = 999424
DT_NCH = DT_FULL // DT_CH
DT_TAIL = 640


def _sc_detile(Pt, Qt, Ut, Vt):
    mesh = plsc.VectorSubcoreMesh(core_axis_name="c", subcore_axis_name="s")
    out = jax.ShapeDtypeStruct((NF * NUP,), jnp.float32)

    @functools.partial(
        pl.kernel,
        mesh=mesh,
        out_type=(out, out, out, out),
        compiler_params=pltpu.CompilerParams(
            needs_layout_passes=False, disable_bounds_checks=True),
        scratch_types=[
            pltpu.VMEM((2, 8, DT_CH), jnp.float32),
            pltpu.SemaphoreType.DMA,
            pltpu.SemaphoreType.DMA,
        ],
    )
    def k(p_hbm, q_hbm, u_hbm, v_hbm, op_hbm, oq_hbm, ou_hbm, ov_hbm,
          buf, sin, sout):
        wid = lax.axis_index("s") * NC + lax.axis_index("c")
        sub = wid % 8
        f0 = (sub // 2) * 8
        h = sub % 2
        c0 = h * (DT_NCH // 2)

        for t, (tbl, dst) in enumerate(
                [(p_hbm, op_hbm), (q_hbm, oq_hbm),
                 (u_hbm, ou_hbm), (v_hbm, ov_hbm)]):
            @pl.when(wid // 8 == t)
            def _(tbl=tbl, dst=dst):
                def fire_in(c, cb):
                    pltpu.async_copy(
                        tbl.at[pl.ds(f0, 8), pl.ds(c * DT_CH, DT_CH)],
                        buf.at[cb], sin)

                def wait_in():
                    pltpu.make_async_copy(
                        tbl.at[pl.ds(0, 8), pl.ds(0, DT_CH)],
                        buf.at[0], sin).wait()

                def fire_outs(c, cb):
                    for s in range(8):
                        pltpu.async_copy(
                            buf.at[cb, s],
                            dst.at[pl.ds((f0 + s) * NUP + c * DT_CH, DT_CH)],
                            sout)

                def wait_outs():
                    for s in range(8):
                        pltpu.make_async_copy(
                            buf.at[0, s],
                            dst.at[pl.ds(0, DT_CH)], sout).wait()

                fire_in(c0, 0)

                @pl.loop(0, DT_NCH // 2)
                def _(j):
                    c = c0 + j
                    cb = j % 2

                    @pl.when(j > 0)
                    def _():
                        wait_outs()

                    @pl.when(j < DT_NCH // 2 - 1)
                    def _():
                        fire_in(c + 1, (j + 1) % 2)

                    wait_in()
                    fire_outs(c, cb)

                wait_outs()

                @pl.when(h == 1)
                def _():
                    u0 = pl.multiple_of(wid * 0 + DT_FULL, 128)
                    pltpu.sync_copy(
                        tbl.at[pl.ds(f0, 8), pl.ds(u0, DT_TAIL)],
                        buf.at[0, :, pl.ds(0, DT_TAIL)])
                    for s in range(8):
                        pltpu.sync_copy(
                            buf.at[0, s, pl.ds(0, DT_TAIL)],
                            dst.at[pl.ds((f0 + s) * NUP + DT_FULL, DT_TAIL)])

    return k(Pt, Qt, Ut, Vt)


def _sc_gather_t(Pl, Ql, Ul, Vl, user_id, item_id):
    mesh = plsc.VectorSubcoreMesh(core_axis_name="c", subcore_axis_name="s")
    out = jax.ShapeDtypeStruct((NF, BATCH), jnp.float32)

    @functools.partial(
        pl.kernel,
        mesh=mesh,
        out_type=(out, out, out, out),
        compiler_params=pltpu.CompilerParams(
            use_tc_tiling_on_sc=False, needs_layout_passes=False),
        scratch_types=[
            pltpu.VMEM((B_PER_W,), jnp.int32),
            pltpu.VMEM((B_PER_W,), jnp.int32),
            pltpu.VMEM((B_PER_W,), jnp.int32),
            pltpu.VMEM((B_PER_W,), jnp.int32),
            pltpu.VMEM((NF, B_PER_W), jnp.float32),
            pltpu.VMEM((NF, B_PER_W), jnp.float32),
            pltpu.VMEM((NF, B_PER_W), jnp.float32),
            pltpu.VMEM((NF, B_PER_W), jnp.float32),
            pltpu.SemaphoreType.DMA,
            pltpu.SemaphoreType.DMA,
            pltpu.SemaphoreType.DMA,
            pltpu.SemaphoreType.DMA,
        ],
    )
    def k(p_hbm, q_hbm, u_hbm, v_hbm, iu_hbm, ii_hbm,
          pmf_hbm, qmf_hbm, pml_hbm, qml_hbm,
          iu_v, ii_v, eu_v, ei_v, pv, qv, uv, vv, sp, sq, su, sv):
        wid = lax.axis_index("s") * NC + lax.axis_index("c")
        base = wid * B_PER_W
        pltpu.sync_copy(iu_hbm.at[pl.ds(base, B_PER_W)], iu_v)
        pltpu.sync_copy(ii_hbm.at[pl.ds(base, B_PER_W)], ii_v)

        @pl.loop(0, NF)
        def _(f):
            off = f * NUP

            @pl.loop(0, B_PER_W // 16)
            def _(cc):
                s = pl.ds(cc * 16, 16)
                eu_v[s] = iu_v[s] + off
                ei_v[s] = ii_v[s] + off

            pltpu.async_copy(p_hbm.at[eu_v], pv.at[f], sp)
            pltpu.async_copy(q_hbm.at[ei_v], qv.at[f], sq)
            pltpu.async_copy(u_hbm.at[eu_v], uv.at[f], su)
            pltpu.async_copy(v_hbm.at[ei_v], vv.at[f], sv)
            pltpu.make_async_copy(p_hbm.at[eu_v], pv.at[f], sp).wait()
            pltpu.make_async_copy(q_hbm.at[ei_v], qv.at[f], sq).wait()
            pltpu.make_async_copy(u_hbm.at[eu_v], uv.at[f], su).wait()
            pltpu.make_async_copy(v_hbm.at[ei_v], vv.at[f], sv).wait()

        pltpu.sync_copy(pv, pmf_hbm.at[:, pl.ds(base, B_PER_W)])
        pltpu.sync_copy(qv, qmf_hbm.at[:, pl.ds(base, B_PER_W)])
        pltpu.sync_copy(uv, pml_hbm.at[:, pl.ds(base, B_PER_W)])
        pltpu.sync_copy(vv, qml_hbm.at[:, pl.ds(base, B_PER_W)])

    return k(Pl, Ql, Ul, Vl, user_id, item_id)


def _dotT(w, x):
    return lax.dot_general(w, x, (((0,), (0,)), ((), ())),
                           preferred_element_type=jnp.float32)


def _tc_mlp_body(pmf_ref, qmf_ref, pml_ref, qml_ref,
                 w0_ref, b0_ref, w1_ref, b1_ref, w2_ref, b2_ref,
                 wp_ref, bp_ref, out_ref):
    h = (_dotT(w0_ref[:NF, :], pml_ref[...])
         + _dotT(w0_ref[NF:, :], qml_ref[...])
         + b0_ref[...].T)
    h = jnp.maximum(h, 0.0)
    h = _dotT(w1_ref[...], h) + b1_ref[...].T
    h = jnp.maximum(h, 0.0)
    h = _dotT(w2_ref[...], h) + b2_ref[...].T
    h = jnp.maximum(h, 0.0)
    gmf = pmf_ref[...] * qmf_ref[...]
    out = (_dotT(wp_ref[:NF, :], gmf)
           + _dotT(wp_ref[NF:, :], h)
           + bp_ref[...].T)
    out_ref[...] = out


def _tc_mlp(pmf, qmf, pml, qml, W0, b0, W1, b1, W2, b2, Wp, bp):
    blk = 4096
    grid = (BATCH // blk,)
    in_col = pl.BlockSpec((NF, blk), lambda i: (0, i))
    full = lambda a: pl.BlockSpec(a.shape, lambda i: (0,) * a.ndim)
    return pl.pallas_call(
        _tc_mlp_body,
        grid=grid,
        in_specs=[in_col, in_col, in_col, in_col,
                  full(W0), full(b0), full(W1), full(b1),
                  full(W2), full(b2), full(Wp), full(bp)],
        out_specs=pl.BlockSpec((1, blk), lambda i: (0, i)),
        out_shape=jax.ShapeDtypeStruct((1, BATCH), jnp.float32),
    )(pmf, qmf, pml, qml, W0, b0, W1, b1, W2, b2, Wp, bp)


def kernel(user_id, item_id, P, Q, U, V, W0, b0, W1, b1, W2, b2, Wp, bp):
    Pl, Ql, Ul, Vl = _sc_detile(P.T, Q.T, U.T, V.T)
    pmf, qmf, pml, qml = _sc_gather_t(Pl, Ql, Ul, Vl, user_id, item_id)
    out = _tc_mlp(pmf, qmf, pml, qml,
                  W0, b0.reshape(1, -1), W1, b1.reshape(1, -1),
                  W2, b2.reshape(1, -1), Wp, bp.reshape(1, -1))
    return out.reshape(BATCH)


# speedup vs baseline: 32.8112x; 1.0254x over previous
import functools

import jax
import jax.numpy as jnp
from jax import lax
from jax.experimental import pallas as pl
from jax.experimental.pallas import tpu as pltpu
from jax.experimental.pallas import tpu_sc as plsc

BATCH = 16384
NF = 32
NC, NS = 2, 16
NW = NC * NS
B_PER_W = BATCH // NW
IC = 128
N_IC = B_PER_W // IC


NU = 1000000
NUP = 1000064
DT_CH = 4096
DT_FULL = 999424
DT_NCH = DT_FULL // DT_CH
DT_TAIL = 640


def _sc_detile(Pt, Qt, Ut, Vt):
    mesh = plsc.VectorSubcoreMesh(core_axis_name="c", subcore_axis_name="s")
    out = jax.ShapeDtypeStruct((NF * NUP,), jnp.float32)

    @functools.partial(
        pl.kernel,
        mesh=mesh,
        out_type=(out, out, out, out),
        compiler_params=pltpu.CompilerParams(
            needs_layout_passes=False, disable_bounds_checks=True),
        scratch_types=[
            pltpu.VMEM((2, 8, DT_CH), jnp.float32),
            pltpu.SemaphoreType.DMA,
            pltpu.SemaphoreType.DMA,
        ],
    )
    def k(p_hbm, q_hbm, u_hbm, v_hbm, op_hbm, oq_hbm, ou_hbm, ov_hbm,
          buf, sin, sout):
        wid = lax.axis_index("s") * NC + lax.axis_index("c")
        sub = wid % 8
        f0 = (sub // 2) * 8
        h = sub % 2
        c0 = h * (DT_NCH // 2)

        for t, (tbl, dst) in enumerate(
                [(p_hbm, op_hbm), (q_hbm, oq_hbm),
                 (u_hbm, ou_hbm), (v_hbm, ov_hbm)]):
            @pl.when(wid // 8 == t)
            def _(tbl=tbl, dst=dst):
                def fire_in(c, cb):
                    pltpu.async_copy(
                        tbl.at[pl.ds(f0, 8), pl.ds(c * DT_CH, DT_CH)],
                        buf.at[cb], sin)

                def wait_in():
                    pltpu.make_async_copy(
                        tbl.at[pl.ds(0, 8), pl.ds(0, DT_CH)],
                        buf.at[0], sin).wait()

                def fire_outs(c, cb):
                    for s in range(8):
                        pltpu.async_copy(
                            buf.at[cb, s],
                            dst.at[pl.ds((f0 + s) * NUP + c * DT_CH, DT_CH)],
                            sout)

                def wait_outs():
                    for s in range(8):
                        pltpu.make_async_copy(
                            buf.at[0, s],
                            dst.at[pl.ds(0, DT_CH)], sout).wait()

                fire_in(c0, 0)

                @pl.loop(0, DT_NCH // 2)
                def _(j):
                    c = c0 + j
                    cb = j % 2

                    @pl.when(j > 0)
                    def _():
                        wait_outs()

                    @pl.when(j < DT_NCH // 2 - 1)
                    def _():
                        fire_in(c + 1, (j + 1) % 2)

                    wait_in()
                    fire_outs(c, cb)

                wait_outs()

                @pl.when(h == 1)
                def _():
                    u0 = pl.multiple_of(wid * 0 + DT_FULL, 128)
                    pltpu.sync_copy(
                        tbl.at[pl.ds(f0, 8), pl.ds(u0, DT_TAIL)],
                        buf.at[0, :, pl.ds(0, DT_TAIL)])
                    for s in range(8):
                        pltpu.sync_copy(
                            buf.at[0, s, pl.ds(0, DT_TAIL)],
                            dst.at[pl.ds((f0 + s) * NUP + DT_FULL, DT_TAIL)])

    return k(Pt, Qt, Ut, Vt)


def _sc_gather_t(Pl, Ql, Ul, Vl, user_id, item_id):
    mesh = plsc.VectorSubcoreMesh(core_axis_name="c", subcore_axis_name="s")
    out = jax.ShapeDtypeStruct((NF, BATCH), jnp.float32)

    @functools.partial(
        pl.kernel,
        mesh=mesh,
        out_type=(out, out, out, out),
        compiler_params=pltpu.CompilerParams(
            use_tc_tiling_on_sc=False, needs_layout_passes=False),
        scratch_types=[
            pltpu.VMEM((B_PER_W,), jnp.int32),
            pltpu.VMEM((B_PER_W,), jnp.int32),
            pltpu.VMEM((2, B_PER_W), jnp.int32),
            pltpu.VMEM((2, B_PER_W), jnp.int32),
            pltpu.VMEM((NF, B_PER_W), jnp.float32),
            pltpu.VMEM((NF, B_PER_W), jnp.float32),
            pltpu.VMEM((NF, B_PER_W), jnp.float32),
            pltpu.VMEM((NF, B_PER_W), jnp.float32),
            pltpu.SemaphoreType.DMA,
            pltpu.SemaphoreType.DMA,
            pltpu.SemaphoreType.DMA,
            pltpu.SemaphoreType.DMA,
        ],
    )
    def k(p_hbm, q_hbm, u_hbm, v_hbm, iu_hbm, ii_hbm,
          pmf_hbm, qmf_hbm, pml_hbm, qml_hbm,
          iu_v, ii_v, eu_v, ei_v, pv, qv, uv, vv, sp, sq, su, sv):
        wid = lax.axis_index("s") * NC + lax.axis_index("c")
        base = wid * B_PER_W
        pltpu.sync_copy(iu_hbm.at[pl.ds(base, B_PER_W)], iu_v)
        pltpu.sync_copy(ii_hbm.at[pl.ds(base, B_PER_W)], ii_v)

        def fire(f):
            off = f * NUP
            fb = f % 2

            @pl.loop(0, B_PER_W // 16)
            def _(cc):
                s = pl.ds(cc * 16, 16)
                eu_v[fb, s] = iu_v[s] + off
                ei_v[fb, s] = ii_v[s] + off

            eu = eu_v.at[fb]
            ei = ei_v.at[fb]
            pltpu.async_copy(p_hbm.at[eu], pv.at[f], sp)
            pltpu.async_copy(q_hbm.at[ei], qv.at[f], sq)
            pltpu.async_copy(u_hbm.at[eu], uv.at[f], su)
            pltpu.async_copy(v_hbm.at[ei], vv.at[f], sv)

        def drain(f):
            eu = eu_v.at[f % 2]
            ei = ei_v.at[f % 2]
            pltpu.make_async_copy(p_hbm.at[eu], pv.at[f], sp).wait()
            pltpu.make_async_copy(q_hbm.at[ei], qv.at[f], sq).wait()
            pltpu.make_async_copy(u_hbm.at[eu], uv.at[f], su).wait()
            pltpu.make_async_copy(v_hbm.at[ei], vv.at[f], sv).wait()

        fire(0)

        @pl.loop(1, NF)
        def _(f):
            fire(f)
            drain(f - 1)

        drain(NF - 1)

        pltpu.sync_copy(pv, pmf_hbm.at[:, pl.ds(base, B_PER_W)])
        pltpu.sync_copy(qv, qmf_hbm.at[:, pl.ds(base, B_PER_W)])
        pltpu.sync_copy(uv, pml_hbm.at[:, pl.ds(base, B_PER_W)])
        pltpu.sync_copy(vv, qml_hbm.at[:, pl.ds(base, B_PER_W)])

    return k(Pl, Ql, Ul, Vl, user_id, item_id)


def _dotT(w, x):
    return lax.dot_general(w, x, (((0,), (0,)), ((), ())),
                           preferred_element_type=jnp.float32)


def _tc_mlp_body(pmf_ref, qmf_ref, pml_ref, qml_ref,
                 w0_ref, b0_ref, w1_ref, b1_ref, w2_ref, b2_ref,
                 wp_ref, bp_ref, out_ref):
    h = (_dotT(w0_ref[:NF, :], pml_ref[...])
         + _dotT(w0_ref[NF:, :], qml_ref[...])
         + b0_ref[...].T)
    h = jnp.maximum(h, 0.0)
    h = _dotT(w1_ref[...], h) + b1_ref[...].T
    h = jnp.maximum(h, 0.0)
    h = _dotT(w2_ref[...], h) + b2_ref[...].T
    h = jnp.maximum(h, 0.0)
    gmf = pmf_ref[...] * qmf_ref[...]
    out = (_dotT(wp_ref[:NF, :], gmf)
           + _dotT(wp_ref[NF:, :], h)
           + bp_ref[...].T)
    out_ref[...] = out


def _tc_mlp(pmf, qmf, pml, qml, W0, b0, W1, b1, W2, b2, Wp, bp):
    blk = 4096
    grid = (BATCH // blk,)
    in_col = pl.BlockSpec((NF, blk), lambda i: (0, i))
    full = lambda a: pl.BlockSpec(a.shape, lambda i: (0,) * a.ndim)
    return pl.pallas_call(
        _tc_mlp_body,
        grid=grid,
        in_specs=[in_col, in_col, in_col, in_col,
                  full(W0), full(b0), full(W1), full(b1),
                  full(W2), full(b2), full(Wp), full(bp)],
        out_specs=pl.BlockSpec((1, blk), lambda i: (0, i)),
        out_shape=jax.ShapeDtypeStruct((1, BATCH), jnp.float32),
    )(pmf, qmf, pml, qml, W0, b0, W1, b1, W2, b2, Wp, bp)


def kernel(user_id, item_id, P, Q, U, V, W0, b0, W1, b1, W2, b2, Wp, bp):
    Pl, Ql, Ul, Vl = _sc_detile(P.T, Q.T, U.T, V.T)
    pmf, qmf, pml, qml = _sc_gather_t(Pl, Ql, Ul, Vl, user_id, item_id)
    out = _tc_mlp(pmf, qmf, pml, qml,
                  W0, b0.reshape(1, -1), W1, b1.reshape(1, -1),
                  W2, b2.reshape(1, -1), Wp, bp.reshape(1, -1))
    return out.reshape(BATCH)


# speedup vs baseline: 32.9323x; 1.0037x over previous
import functools

import jax
import jax.numpy as jnp
from jax import lax
from jax.experimental import pallas as pl
from jax.experimental.pallas import tpu as pltpu
from jax.experimental.pallas import tpu_sc as plsc

BATCH = 16384
NF = 32
NC, NS = 2, 16
NW = NC * NS
B_PER_W = BATCH // NW
IC = 128
N_IC = B_PER_W // IC


NU = 1000000
NUP = 1000064
DT_CH = 7808
DT_FULL = 999424
DT_NCH = DT_FULL // DT_CH
DT_TAIL = 640


def _sc_detile(Pt, Qt, Ut, Vt):
    mesh = plsc.VectorSubcoreMesh(core_axis_name="c", subcore_axis_name="s")
    out = jax.ShapeDtypeStruct((NF * NUP,), jnp.float32)

    @functools.partial(
        pl.kernel,
        mesh=mesh,
        out_type=(out, out, out, out),
        compiler_params=pltpu.CompilerParams(
            needs_layout_passes=False, disable_bounds_checks=True),
        scratch_types=[
            pltpu.VMEM((2, 8, DT_CH), jnp.float32),
            pltpu.SemaphoreType.DMA,
            pltpu.SemaphoreType.DMA,
        ],
    )
    def k(p_hbm, q_hbm, u_hbm, v_hbm, op_hbm, oq_hbm, ou_hbm, ov_hbm,
          buf, sin, sout):
        wid = lax.axis_index("s") * NC + lax.axis_index("c")
        sub = wid % 8
        f0 = (sub // 2) * 8
        h = sub % 2
        c0 = h * (DT_NCH // 2)

        for t, (tbl, dst) in enumerate(
                [(p_hbm, op_hbm), (q_hbm, oq_hbm),
                 (u_hbm, ou_hbm), (v_hbm, ov_hbm)]):
            @pl.when(wid // 8 == t)
            def _(tbl=tbl, dst=dst):
                def fire_in(c, cb):
                    pltpu.async_copy(
                        tbl.at[pl.ds(f0, 8), pl.ds(c * DT_CH, DT_CH)],
                        buf.at[cb], sin)

                def wait_in():
                    pltpu.make_async_copy(
                        tbl.at[pl.ds(0, 8), pl.ds(0, DT_CH)],
                        buf.at[0], sin).wait()

                def fire_outs(c, cb):
                    for s in range(8):
                        pltpu.async_copy(
                            buf.at[cb, s],
                            dst.at[pl.ds((f0 + s) * NUP + c * DT_CH, DT_CH)],
                            sout)

                def wait_outs():
                    for s in range(8):
                        pltpu.make_async_copy(
                            buf.at[0, s],
                            dst.at[pl.ds(0, DT_CH)], sout).wait()

                fire_in(c0, 0)

                @pl.loop(0, DT_NCH // 2)
                def _(j):
                    c = c0 + j
                    cb = j % 2

                    @pl.when(j > 0)
                    def _():
                        wait_outs()

                    @pl.when(j < DT_NCH // 2 - 1)
                    def _():
                        fire_in(c + 1, (j + 1) % 2)

                    wait_in()
                    fire_outs(c, cb)

                wait_outs()

                @pl.when(h == 1)
                def _():
                    u0 = pl.multiple_of(wid * 0 + DT_FULL, 128)
                    pltpu.sync_copy(
                        tbl.at[pl.ds(f0, 8), pl.ds(u0, DT_TAIL)],
                        buf.at[0, :, pl.ds(0, DT_TAIL)])
                    for s in range(8):
                        pltpu.sync_copy(
                            buf.at[0, s, pl.ds(0, DT_TAIL)],
                            dst.at[pl.ds((f0 + s) * NUP + DT_FULL, DT_TAIL)])

    return k(Pt, Qt, Ut, Vt)


def _sc_gather_t(Pl, Ql, Ul, Vl, user_id, item_id):
    mesh = plsc.VectorSubcoreMesh(core_axis_name="c", subcore_axis_name="s")
    out = jax.ShapeDtypeStruct((NF, BATCH), jnp.float32)

    @functools.partial(
        pl.kernel,
        mesh=mesh,
        out_type=(out, out, out, out),
        compiler_params=pltpu.CompilerParams(
            use_tc_tiling_on_sc=False, needs_layout_passes=False),
        scratch_types=[
            pltpu.VMEM((B_PER_W,), jnp.int32),
            pltpu.VMEM((B_PER_W,), jnp.int32),
            pltpu.VMEM((2, B_PER_W), jnp.int32),
            pltpu.VMEM((2, B_PER_W), jnp.int32),
            pltpu.VMEM((NF, B_PER_W), jnp.float32),
            pltpu.VMEM((NF, B_PER_W), jnp.float32),
            pltpu.VMEM((NF, B_PER_W), jnp.float32),
            pltpu.VMEM((NF, B_PER_W), jnp.float32),
            pltpu.SemaphoreType.DMA,
            pltpu.SemaphoreType.DMA,
            pltpu.SemaphoreType.DMA,
            pltpu.SemaphoreType.DMA,
        ],
    )
    def k(p_hbm, q_hbm, u_hbm, v_hbm, iu_hbm, ii_hbm,
          pmf_hbm, qmf_hbm, pml_hbm, qml_hbm,
          iu_v, ii_v, eu_v, ei_v, pv, qv, uv, vv, sp, sq, su, sv):
        wid = lax.axis_index("s") * NC + lax.axis_index("c")
        base = wid * B_PER_W
        pltpu.sync_copy(iu_hbm.at[pl.ds(base, B_PER_W)], iu_v)
        pltpu.sync_copy(ii_hbm.at[pl.ds(base, B_PER_W)], ii_v)

        def fire(f):
            off = f * NUP
            fb = f % 2

            @pl.loop(0, B_PER_W // 16)
            def _(cc):
                s = pl.ds(cc * 16, 16)
                eu_v[fb, s] = iu_v[s] + off
                ei_v[fb, s] = ii_v[s] + off

            eu = eu_v.at[fb]
            ei = ei_v.at[fb]
            pltpu.async_copy(p_hbm.at[eu], pv.at[f], sp)
            pltpu.async_copy(q_hbm.at[ei], qv.at[f], sq)
            pltpu.async_copy(u_hbm.at[eu], uv.at[f], su)
            pltpu.async_copy(v_hbm.at[ei], vv.at[f], sv)

        def drain(f):
            eu = eu_v.at[f % 2]
            ei = ei_v.at[f % 2]
            pltpu.make_async_copy(p_hbm.at[eu], pv.at[f], sp).wait()
            pltpu.make_async_copy(q_hbm.at[ei], qv.at[f], sq).wait()
            pltpu.make_async_copy(u_hbm.at[eu], uv.at[f], su).wait()
            pltpu.make_async_copy(v_hbm.at[ei], vv.at[f], sv).wait()

        fire(0)

        @pl.loop(1, NF)
        def _(f):
            fire(f)
            drain(f - 1)

        drain(NF - 1)

        pltpu.sync_copy(pv, pmf_hbm.at[:, pl.ds(base, B_PER_W)])
        pltpu.sync_copy(qv, qmf_hbm.at[:, pl.ds(base, B_PER_W)])
        pltpu.sync_copy(uv, pml_hbm.at[:, pl.ds(base, B_PER_W)])
        pltpu.sync_copy(vv, qml_hbm.at[:, pl.ds(base, B_PER_W)])

    return k(Pl, Ql, Ul, Vl, user_id, item_id)


def _dotT(w, x):
    return lax.dot_general(w, x, (((0,), (0,)), ((), ())),
                           preferred_element_type=jnp.float32)


def _tc_mlp_body(pmf_ref, qmf_ref, pml_ref, qml_ref,
                 w0_ref, b0_ref, w1_ref, b1_ref, w2_ref, b2_ref,
                 wp_ref, bp_ref, out_ref):
    h = (_dotT(w0_ref[:NF, :], pml_ref[...])
         + _dotT(w0_ref[NF:, :], qml_ref[...])
         + b0_ref[...].T)
    h = jnp.maximum(h, 0.0)
    h = _dotT(w1_ref[...], h) + b1_ref[...].T
    h = jnp.maximum(h, 0.0)
    h = _dotT(w2_ref[...], h) + b2_ref[...].T
    h = jnp.maximum(h, 0.0)
    gmf = pmf_ref[...] * qmf_ref[...]
    out = (_dotT(wp_ref[:NF, :], gmf)
           + _dotT(wp_ref[NF:, :], h)
           + bp_ref[...].T)
    out_ref[...] = out


def _tc_mlp(pmf, qmf, pml, qml, W0, b0, W1, b1, W2, b2, Wp, bp):
    blk = 4096
    grid = (BATCH // blk,)
    in_col = pl.BlockSpec((NF, blk), lambda i: (0, i))
    full = lambda a: pl.BlockSpec(a.shape, lambda i: (0,) * a.ndim)
    return pl.pallas_call(
        _tc_mlp_body,
        grid=grid,
        in_specs=[in_col, in_col, in_col, in_col,
                  full(W0), full(b0), full(W1), full(b1),
                  full(W2), full(b2), full(Wp), full(bp)],
        out_specs=pl.BlockSpec((1, blk), lambda i: (0, i)),
        out_shape=jax.ShapeDtypeStruct((1, BATCH), jnp.float32),
    )(pmf, qmf, pml, qml, W0, b0, W1, b1, W2, b2, Wp, bp)


def kernel(user_id, item_id, P, Q, U, V, W0, b0, W1, b1, W2, b2, Wp, bp):
    Pl, Ql, Ul, Vl = _sc_detile(P.T, Q.T, U.T, V.T)
    pmf, qmf, pml, qml = _sc_gather_t(Pl, Ql, Ul, Vl, user_id, item_id)
    out = _tc_mlp(pmf, qmf, pml, qml,
                  W0, b0.reshape(1, -1), W1, b1.reshape(1, -1),
                  W2, b2.reshape(1, -1), Wp, bp.reshape(1, -1))
    return out.reshape(BATCH)
